# baseline (speedup 1.0000x reference)
import functools

import jax
import jax.numpy as jnp
from jax import lax
from jax.experimental import pallas as pl
from jax.experimental.pallas import tpu as pltpu
from jax.experimental.pallas import tpu_sc as plsc


def _make_dma_swap_body(n_chunks, win_rows):
    def body(t_ref, mem, win, idx, ind, omem, oidx, sem):
        rows = mem.shape[0] // n_chunks
        irows = idx.shape[0] // n_chunks
        cps = []
        for k in range(n_chunks):
            cp = pltpu.make_async_copy(
                mem.at[pl.ds(k * rows, rows), :],
                omem.at[pl.ds(k * rows, rows), :], sem)
            cp.start()
            cps.append(cp)
            cp = pltpu.make_async_copy(
                idx.at[pl.ds(k * irows, irows)],
                oidx.at[pl.ds(k * irows, irows)], sem)
            cp.start()
            cps.append(cp)
        for cp in cps:
            cp.wait()
        start = t_ref[0] * win_rows
        w1 = pltpu.make_async_copy(win, omem.at[pl.ds(start, win_rows), :], sem)
        w2 = pltpu.make_async_copy(ind, oidx.at[pl.ds(start, win_rows)], sem)
        w1.start()
        w2.start()
        w1.wait()
        w2.wait()
    return body


def _queue_update(mem, win, idx, ind, t, n_chunks):
    any_spec = pl.BlockSpec(memory_space=pl.ANY)
    return pl.pallas_call(
        _make_dma_swap_body(n_chunks, win.shape[0]),
        in_specs=[
            pl.BlockSpec(memory_space=pltpu.SMEM),
            any_spec, any_spec, any_spec, any_spec,
        ],
        out_specs=[any_spec, any_spec],
        out_shape=[
            jax.ShapeDtypeStruct(mem.shape, mem.dtype),
            jax.ShapeDtypeStruct(idx.shape, idx.dtype),
        ],
        scratch_shapes=[pltpu.SemaphoreType.DMA],
    )(t, mem, win, idx, ind)


def _sc_gather(table, idx):
    info = plsc.get_sparse_core_info()
    nc = info.num_cores
    b = idx.shape[0]
    d = table.shape[1]
    bpw = b // nc
    mesh = plsc.ScalarSubcoreMesh(axis_name="core", num_cores=nc)

    @functools.partial(
        pl.kernel,
        out_type=jax.ShapeDtypeStruct((b, d), table.dtype),
        mesh=mesh,
        scratch_types=[
            pltpu.SMEM((bpw,), jnp.int32),
            pltpu.SemaphoreType.DMA,
            pltpu.SemaphoreType.DMA,
        ],
    )
    def k(table_hbm, idx_hbm, out_hbm, idx_s, sem_i, sem):
        cid = lax.axis_index("core")
        base = cid * bpw
        pltpu.async_copy(idx_hbm.at[pl.ds(base, bpw)], idx_s, sem_i).wait()

        @pl.loop(0, bpw)
        def _(r):
            j = idx_s[r]
            pltpu.make_async_copy(
                table_hbm.at[pl.ds(j, 1), :],
                out_hbm.at[pl.ds(base + r, 1), :],
                sem,
            ).start()

        pltpu.make_async_copy(
            table_hbm.at[pl.ds(0, bpw), :],
            out_hbm.at[pl.ds(base, bpw), :],
            sem,
        ).wait()

    return k(table, idx)


def kernel(mem_ref, mem_pos, Y_ref, Z, indices, idx_ref_buf, idx_pos_buf,
           pos_sampled_idx, step_rel):
    B, d = Y_ref.shape
    R = mem_ref.shape[0]
    P = mem_pos.shape[0]

    step = jnp.asarray(step_rel, jnp.int32)
    t_ref = jnp.reshape(((step * B) % R) // B, (1,))
    t_pos = jnp.reshape(((step * B) % P) // B, (1,))

    mem_pos_new, idx_pos_new = _queue_update(mem_pos, Z, idx_pos_buf, indices,
                                             t_pos, n_chunks=4)

    mem_ref_new, idx_ref_new = _queue_update(mem_ref, Y_ref, idx_ref_buf,
                                             indices, t_ref, n_chunks=16)

    Z_pseudo = _sc_gather(mem_pos_new, pos_sampled_idx)

    return mem_ref_new, idx_ref_new, mem_pos_new, idx_pos_new, Z_pseudo


# speedup vs baseline: 1.0826x; 1.0826x over previous
import functools

import jax
import jax.numpy as jnp
from jax import lax
from jax.experimental import pallas as pl
from jax.experimental.pallas import tpu as pltpu
from jax.experimental.pallas import tpu_sc as plsc


def _copy_swap_body(t_ref, mem_ref, win_ref, idx_ref, ind_ref, omem_ref, oidx_ref):
    i = pl.program_id(0)
    t = t_ref[0]

    @pl.when(i == t)
    def _():
        omem_ref[...] = win_ref[...]
        oidx_ref[...] = ind_ref[...]

    @pl.when(i != t)
    def _():
        omem_ref[...] = mem_ref[...]
        oidx_ref[...] = idx_ref[...]


def _queue_update(mem, win, idx, ind, t):
    b, d = win.shape
    n = mem.shape[0] // b
    return pl.pallas_call(
        _copy_swap_body,
        grid=(n,),
        in_specs=[
            pl.BlockSpec(memory_space=pltpu.SMEM),
            pl.BlockSpec((b, d), lambda i: (i, 0)),
            pl.BlockSpec((b, d), lambda i: (0, 0)),
            pl.BlockSpec((b,), lambda i: (i,)),
            pl.BlockSpec((b,), lambda i: (0,)),
        ],
        out_specs=[
            pl.BlockSpec((b, d), lambda i: (i, 0)),
            pl.BlockSpec((b,), lambda i: (i,)),
        ],
        out_shape=[
            jax.ShapeDtypeStruct(mem.shape, mem.dtype),
            jax.ShapeDtypeStruct(idx.shape, idx.dtype),
        ],
        compiler_params=pltpu.CompilerParams(dimension_semantics=("parallel",)),
    )(t, mem, win, idx, ind)


def _sc_bulk_copy(x, n_sub=8):
    info = plsc.get_sparse_core_info()
    nc, ns = info.num_cores, info.num_subcores
    nw = nc * ns
    rows_w = x.shape[0] // nw
    rows_c = rows_w // n_sub
    mesh = plsc.VectorSubcoreMesh(core_axis_name="c", subcore_axis_name="s")

    @functools.partial(
        pl.kernel,
        out_type=jax.ShapeDtypeStruct(x.shape, x.dtype),
        mesh=mesh,
        scratch_types=[pltpu.SemaphoreType.DMA],
    )
    def k(x_hbm, o_hbm, sem):
        wid = lax.axis_index("s") * nc + lax.axis_index("c")
        base = wid * rows_w
        for j in range(n_sub):
            pltpu.make_async_copy(
                x_hbm.at[pl.ds(base + j * rows_c, rows_c), :],
                o_hbm.at[pl.ds(base + j * rows_c, rows_c), :],
                sem,
            ).start()
        pltpu.make_async_copy(
            x_hbm.at[pl.ds(base, rows_w), :],
            o_hbm.at[pl.ds(base, rows_w), :],
            sem,
        ).wait()

    return k(x)


def _window_write_body(t_ref, win_ref, o_ref):
    o_ref[...] = win_ref[...]


def _window_write(dst, win, t):
    b, d = win.shape
    grid_spec = pltpu.PrefetchScalarGridSpec(
        num_scalar_prefetch=1,
        grid=(1,),
        in_specs=[
            pl.BlockSpec(memory_space=pl.ANY),
            pl.BlockSpec((b, d), lambda i, t: (0, 0)),
        ],
        out_specs=pl.BlockSpec((b, d), lambda i, t: (t[0], 0)),
    )

    def body(t_ref, dst_ref, win_ref, o_ref):
        o_ref[...] = win_ref[...]

    return pl.pallas_call(
        body,
        grid_spec=grid_spec,
        out_shape=jax.ShapeDtypeStruct(dst.shape, dst.dtype),
        input_output_aliases={1: 0},
    )(t, dst, win)


def _sc_gather(table, idx):
    info = plsc.get_sparse_core_info()
    nc = info.num_cores
    b = idx.shape[0]
    d = table.shape[1]
    bpw = b // nc
    mesh = plsc.ScalarSubcoreMesh(axis_name="core", num_cores=nc)

    @functools.partial(
        pl.kernel,
        out_type=jax.ShapeDtypeStruct((b, d), table.dtype),
        mesh=mesh,
        scratch_types=[
            pltpu.SMEM((bpw,), jnp.int32),
            pltpu.SemaphoreType.DMA,
            pltpu.SemaphoreType.DMA,
        ],
    )
    def k(table_hbm, idx_hbm, out_hbm, idx_s, sem_i, sem):
        cid = lax.axis_index("core")
        base = cid * bpw
        pltpu.async_copy(idx_hbm.at[pl.ds(base, bpw)], idx_s, sem_i).wait()

        @pl.loop(0, bpw)
        def _(r):
            j = idx_s[r]
            pltpu.make_async_copy(
                table_hbm.at[pl.ds(j, 1), :],
                out_hbm.at[pl.ds(base + r, 1), :],
                sem,
            ).start()

        pltpu.make_async_copy(
            table_hbm.at[pl.ds(0, bpw), :],
            out_hbm.at[pl.ds(base, bpw), :],
            sem,
        ).wait()

    return k(table, idx)


def kernel(mem_ref, mem_pos, Y_ref, Z, indices, idx_ref_buf, idx_pos_buf,
           pos_sampled_idx, step_rel):
    B, d = Y_ref.shape
    R = mem_ref.shape[0]
    P = mem_pos.shape[0]

    step = jnp.asarray(step_rel, jnp.int32)
    t_ref = jnp.reshape(((step * B) % R) // B, (1,))
    t_pos = jnp.reshape(((step * B) % P) // B, (1,))

    mem_ref_copied = _sc_bulk_copy(mem_ref)

    mem_pos_new, idx_pos_new = _queue_update(mem_pos, Z, idx_pos_buf, indices, t_pos)
    idx_ref_new = _queue_update_1d(idx_ref_buf, indices, t_ref)

    Z_pseudo = _sc_gather(mem_pos_new, pos_sampled_idx)

    mem_ref_new = _window_write(mem_ref_copied, Y_ref, t_ref)

    return mem_ref_new, idx_ref_new, mem_pos_new, idx_pos_new, Z_pseudo


def _copy_swap_1d_body(t_ref, idx_ref, ind_ref, oidx_ref):
    i = pl.program_id(0)
    t = t_ref[0]

    @pl.when(i == t)
    def _():
        oidx_ref[...] = ind_ref[...]

    @pl.when(i != t)
    def _():
        oidx_ref[...] = idx_ref[...]


def _queue_update_1d(idx, ind, t):
    b = ind.shape[0]
    n = idx.shape[0] // b
    return pl.pallas_call(
        _copy_swap_1d_body,
        grid=(n,),
        in_specs=[
            pl.BlockSpec(memory_space=pltpu.SMEM),
            pl.BlockSpec((b,), lambda i: (i,)),
            pl.BlockSpec((b,), lambda i: (0,)),
        ],
        out_specs=pl.BlockSpec((b,), lambda i: (i,)),
        out_shape=jax.ShapeDtypeStruct(idx.shape, idx.dtype),
        compiler_params=pltpu.CompilerParams(dimension_semantics=("parallel",)),
    )(t, idx, ind)


# speedup vs baseline: 15.7771x; 14.5740x over previous
import functools

import jax
import jax.numpy as jnp
from jax import lax
from jax.experimental import pallas as pl
from jax.experimental.pallas import tpu as pltpu
from jax.experimental.pallas import tpu_sc as plsc


def _copy_swap_body(t_ref, mem_ref, win_ref, idx_ref, ind_ref, omem_ref, oidx_ref):
    i = pl.program_id(0)
    t = t_ref[0]

    @pl.when(i == t)
    def _():
        omem_ref[...] = win_ref[...]
        oidx_ref[...] = ind_ref[...]

    @pl.when(i != t)
    def _():
        omem_ref[...] = mem_ref[...]
        oidx_ref[...] = idx_ref[...]


def _queue_update(mem, win, idx, ind, t):
    b, d = win.shape
    n = mem.shape[0] // b
    return pl.pallas_call(
        _copy_swap_body,
        grid=(n,),
        in_specs=[
            pl.BlockSpec(memory_space=pltpu.SMEM),
            pl.BlockSpec((b, d), lambda i: (i, 0)),
            pl.BlockSpec((b, d), lambda i: (0, 0)),
            pl.BlockSpec((b,), lambda i: (i,)),
            pl.BlockSpec((b,), lambda i: (0,)),
        ],
        out_specs=[
            pl.BlockSpec((b, d), lambda i: (i, 0)),
            pl.BlockSpec((b,), lambda i: (i,)),
        ],
        out_shape=[
            jax.ShapeDtypeStruct(mem.shape, mem.dtype),
            jax.ShapeDtypeStruct(idx.shape, idx.dtype),
        ],
        compiler_params=pltpu.CompilerParams(dimension_semantics=("parallel",)),
    )(t, mem, win, idx, ind)


def _sc_bulk_copy(x, rows_b=256):
    info = plsc.get_sparse_core_info()
    nc, ns = info.num_cores, info.num_subcores
    nw = nc * ns
    d = x.shape[1]
    rows_w = x.shape[0] // nw
    n = rows_w // rows_b
    assert n % 2 == 0
    mesh = plsc.VectorSubcoreMesh(core_axis_name="c", subcore_axis_name="s")

    @functools.partial(
        pl.kernel,
        out_type=jax.ShapeDtypeStruct(x.shape, x.dtype),
        mesh=mesh,
        scratch_types=[
            pltpu.VMEM((rows_b, d), x.dtype),
            pltpu.VMEM((rows_b, d), x.dtype),
            pltpu.SemaphoreType.DMA,
            pltpu.SemaphoreType.DMA,
        ],
    )
    def k(x_hbm, o_hbm, buf_a, buf_b, sem_a, sem_b):
        wid = lax.axis_index("s") * nc + lax.axis_index("c")
        base = wid * rows_w

        def chunk(ref, g):
            return ref.at[pl.ds(base + g * rows_b, rows_b), :]

        pltpu.make_async_copy(chunk(x_hbm, 0), buf_a, sem_a).start()

        @pl.loop(0, n, step=2)
        def _(g):
            pltpu.make_async_copy(chunk(x_hbm, g + 1), buf_b, sem_b).start()
            pltpu.make_async_copy(chunk(x_hbm, g), buf_a, sem_a).wait()
            pltpu.sync_copy(buf_a, chunk(o_hbm, g))

            @pl.when(g + 2 < n)
            def _():
                pltpu.make_async_copy(chunk(x_hbm, g + 2), buf_a, sem_a).start()

            pltpu.make_async_copy(chunk(x_hbm, g + 1), buf_b, sem_b).wait()
            pltpu.sync_copy(buf_b, chunk(o_hbm, g + 1))

    return k(x)


def _window_write_body(t_ref, win_ref, o_ref):
    o_ref[...] = win_ref[...]


def _window_write(dst, win, t):
    b, d = win.shape
    grid_spec = pltpu.PrefetchScalarGridSpec(
        num_scalar_prefetch=1,
        grid=(1,),
        in_specs=[
            pl.BlockSpec(memory_space=pl.ANY),
            pl.BlockSpec((b, d), lambda i, t: (0, 0)),
        ],
        out_specs=pl.BlockSpec((b, d), lambda i, t: (t[0], 0)),
    )

    def body(t_ref, dst_ref, win_ref, o_ref):
        o_ref[...] = win_ref[...]

    return pl.pallas_call(
        body,
        grid_spec=grid_spec,
        out_shape=jax.ShapeDtypeStruct(dst.shape, dst.dtype),
        input_output_aliases={1: 0},
    )(t, dst, win)


def _sc_gather(table, idx):
    info = plsc.get_sparse_core_info()
    nc = info.num_cores
    b = idx.shape[0]
    d = table.shape[1]
    bpw = b // nc
    mesh = plsc.ScalarSubcoreMesh(axis_name="core", num_cores=nc)

    @functools.partial(
        pl.kernel,
        out_type=jax.ShapeDtypeStruct((b, d), table.dtype),
        mesh=mesh,
        scratch_types=[
            pltpu.SMEM((bpw,), jnp.int32),
            pltpu.SemaphoreType.DMA,
            pltpu.SemaphoreType.DMA,
        ],
    )
    def k(table_hbm, idx_hbm, out_hbm, idx_s, sem_i, sem):
        cid = lax.axis_index("core")
        base = cid * bpw
        pltpu.async_copy(idx_hbm.at[pl.ds(base, bpw)], idx_s, sem_i).wait()

        @pl.loop(0, bpw)
        def _(r):
            j = idx_s[r]
            pltpu.make_async_copy(
                table_hbm.at[pl.ds(j, 1), :],
                out_hbm.at[pl.ds(base + r, 1), :],
                sem,
            ).start()

        pltpu.make_async_copy(
            table_hbm.at[pl.ds(0, bpw), :],
            out_hbm.at[pl.ds(base, bpw), :],
            sem,
        ).wait()

    return k(table, idx)


def kernel(mem_ref, mem_pos, Y_ref, Z, indices, idx_ref_buf, idx_pos_buf,
           pos_sampled_idx, step_rel):
    B, d = Y_ref.shape
    R = mem_ref.shape[0]
    P = mem_pos.shape[0]

    step = jnp.asarray(step_rel, jnp.int32)
    t_ref = jnp.reshape(((step * B) % R) // B, (1,))
    t_pos = jnp.reshape(((step * B) % P) // B, (1,))

    mem_ref_copied = _sc_bulk_copy(mem_ref)

    mem_pos_new, idx_pos_new = _queue_update(mem_pos, Z, idx_pos_buf, indices, t_pos)
    idx_ref_new = _queue_update_1d(idx_ref_buf, indices, t_ref)

    Z_pseudo = _sc_gather(mem_pos_new, pos_sampled_idx)

    mem_ref_new = _window_write(mem_ref_copied, Y_ref, t_ref)

    return mem_ref_new, idx_ref_new, mem_pos_new, idx_pos_new, Z_pseudo


def _copy_swap_1d_body(t_ref, idx_ref, ind_ref, oidx_ref):
    i = pl.program_id(0)
    t = t_ref[0]

    @pl.when(i == t)
    def _():
        oidx_ref[...] = ind_ref[...]

    @pl.when(i != t)
    def _():
        oidx_ref[...] = idx_ref[...]


def _queue_update_1d(idx, ind, t):
    b = ind.shape[0]
    n = idx.shape[0] // b
    return pl.pallas_call(
        _copy_swap_1d_body,
        grid=(n,),
        in_specs=[
            pl.BlockSpec(memory_space=pltpu.SMEM),
            pl.BlockSpec((b,), lambda i: (i,)),
            pl.BlockSpec((b,), lambda i: (0,)),
        ],
        out_specs=pl.BlockSpec((b,), lambda i: (i,)),
        out_shape=jax.ShapeDtypeStruct(idx.shape, idx.dtype),
        compiler_params=pltpu.CompilerParams(dimension_semantics=("parallel",)),
    )(t, idx, ind)
